# baseline (speedup 1.0000x reference)
import jax
import jax.numpy as jnp
from jax.experimental import pallas as pl
from jax.experimental.pallas import tpu as pltpu

_B, _C, _H, _W = 16, 256, 32, 32
_E = 8
_HW = _H * _W


_BB = 16


def _moe_kernel(x_ref, k_ref, wg_ref, bg_ref, we_ref, out_ref):
    ids = jax.lax.broadcasted_iota(jnp.int32, (_BB, _E), 1)

    pooled = jnp.mean(x_ref[:], axis=1)
    gate = jax.lax.dot_general(
        pooled, wg_ref[:], (((1,), (0,)), ((), ())),
        preferred_element_type=jnp.float32,
    ) + bg_ref[:]
    gate = gate - jnp.max(gate, axis=1, keepdims=True)
    eg = jnp.exp(gate)
    sm = eg / jnp.sum(eg, axis=1, keepdims=True)

    w1v = jnp.max(sm, axis=1, keepdims=True)
    i1v = jnp.argmax(sm, axis=1, keepdims=True)
    masked = jnp.where(ids == i1v, -jnp.inf, sm)
    w2v = jnp.max(masked, axis=1, keepdims=True)
    i2v = jnp.argmax(masked, axis=1, keepdims=True)

    for j in range(_BB):
        x_mat = x_ref[j]

        w_comb = (w1v[j, 0] * we_ref[i1v[j, 0]]
                  + w2v[j, 0] * we_ref[i2v[j, 0]])

        y = jax.lax.dot_general(
            x_mat.astype(jnp.bfloat16), w_comb.astype(jnp.bfloat16),
            (((1,), (1,)), ((), ())),
            preferred_element_type=jnp.float32,
        )
        out_ref[j] = x_mat + y * k_ref[j]


@jax.jit
def kernel(x, K, Wg, bg, We):
    bg2 = bg.reshape(1, _E)
    xl = x.reshape(_B, _C, _HW).transpose(0, 2, 1)
    kl = K.reshape(_B, 1, _C)
    grid_spec = pl.GridSpec(
        grid=(_B // _BB,),
        in_specs=[
            pl.BlockSpec((_BB, _HW, _C), lambda b: (b, 0, 0)),
            pl.BlockSpec((_BB, 1, _C), lambda b: (b, 0, 0)),
            pl.BlockSpec((_C, _E), lambda b: (0, 0)),
            pl.BlockSpec((1, _E), lambda b: (0, 0)),
            pl.BlockSpec((_E, _C, _C), lambda b: (0, 0, 0)),
        ],
        out_specs=pl.BlockSpec((_BB, _HW, _C), lambda b: (b, 0, 0)),
    )
    out = pl.pallas_call(
        _moe_kernel,
        grid_spec=grid_spec,
        out_shape=jax.ShapeDtypeStruct((_B, _HW, _C), jnp.float32),
        compiler_params=pltpu.CompilerParams(
            dimension_semantics=("parallel",),
        ),
    )(xl, kl, Wg, bg2, We)
    return out.transpose(0, 2, 1).reshape(_B, _C, _H, _W)


# speedup vs baseline: 1.2301x; 1.2301x over previous
import jax
import jax.numpy as jnp
from jax.experimental import pallas as pl
from jax.experimental.pallas import tpu as pltpu

_B, _C, _H, _W = 16, 256, 32, 32
_E = 8
_HW = _H * _W


_BB = 8


def _moe_kernel(x_ref, k_ref, wg_ref, bg_ref, we_ref, out_ref):
    ids = jax.lax.broadcasted_iota(jnp.int32, (_BB, _E), 1)

    pooled = jnp.mean(x_ref[:], axis=1)
    gate = jax.lax.dot_general(
        pooled, wg_ref[:], (((1,), (0,)), ((), ())),
        preferred_element_type=jnp.float32,
    ) + bg_ref[:]
    gate = gate - jnp.max(gate, axis=1, keepdims=True)
    eg = jnp.exp(gate)
    sm = eg / jnp.sum(eg, axis=1, keepdims=True)

    w1v = jnp.max(sm, axis=1, keepdims=True)
    i1v = jnp.argmax(sm, axis=1, keepdims=True)
    masked = jnp.where(ids == i1v, -jnp.inf, sm)
    w2v = jnp.max(masked, axis=1, keepdims=True)
    i2v = jnp.argmax(masked, axis=1, keepdims=True)

    for j in range(_BB):
        x_mat = x_ref[j]

        w_comb = (w1v[j, 0] * we_ref[i1v[j, 0]]
                  + w2v[j, 0] * we_ref[i2v[j, 0]])

        y = jax.lax.dot_general(
            x_mat.astype(jnp.bfloat16), w_comb.astype(jnp.bfloat16),
            (((1,), (1,)), ((), ())),
            preferred_element_type=jnp.float32,
        )
        out_ref[j] = x_mat + y * k_ref[j]


@jax.jit
def kernel(x, K, Wg, bg, We):
    bg2 = bg.reshape(1, _E)
    xl = x.reshape(_B, _C, _HW).transpose(0, 2, 1)
    kl = K.reshape(_B, 1, _C)
    grid_spec = pl.GridSpec(
        grid=(_B // _BB,),
        in_specs=[
            pl.BlockSpec((_BB, _HW, _C), lambda b: (b, 0, 0)),
            pl.BlockSpec((_BB, 1, _C), lambda b: (b, 0, 0)),
            pl.BlockSpec((_C, _E), lambda b: (0, 0)),
            pl.BlockSpec((1, _E), lambda b: (0, 0)),
            pl.BlockSpec((_E, _C, _C), lambda b: (0, 0, 0)),
        ],
        out_specs=pl.BlockSpec((_BB, _HW, _C), lambda b: (b, 0, 0)),
    )
    out = pl.pallas_call(
        _moe_kernel,
        grid_spec=grid_spec,
        out_shape=jax.ShapeDtypeStruct((_B, _HW, _C), jnp.float32),
        compiler_params=pltpu.CompilerParams(
            dimension_semantics=("parallel",),
        ),
    )(xl, kl, Wg, bg2, We)
    return out.transpose(0, 2, 1).reshape(_B, _C, _H, _W)
